# baseline (speedup 1.0000x reference)
import functools

import jax
import jax.numpy as jnp
from jax import lax
from jax.experimental import pallas as pl
from jax.experimental.pallas import tpu as pltpu
from jax.experimental.pallas import tpu_sc as plsc

N_NODES = 10000
N_EDGES = 320000
D = 128
NCLS = 16

NW = 32
EPW = N_EDGES // NW
CH = 80
NCH = EPW // CH
NG = 5
IB = NCH // NG
NPAD = 10240
RPT = NPAD // 16

RB = 1000
_EPS = 1e-5


def _ln(y, s, b):
    mu = jnp.mean(y, axis=-1, keepdims=True)
    yc = y - mu
    var = jnp.mean(yc * yc, axis=-1, keepdims=True)
    return yc / jnp.sqrt(var + _EPS) * s + b


def _proj_body(h_ref, pw0_ref, pb0_ref, ps0_ref, pB0_ref,
               pw1_ref, pb1_ref, ps1_ref, pB1_ref, x_ref):
    hb = h_ref[...]
    y0 = jnp.dot(hb[:, :64], pw0_ref[...], preferred_element_type=jnp.float32) + pb0_ref[...]
    y1 = jnp.dot(hb[:, 64:], pw1_ref[...], preferred_element_type=jnp.float32) + pb1_ref[...]
    y0 = jnp.maximum(_ln(y0, ps0_ref[...], pB0_ref[...]), 0.0)
    y1 = jnp.maximum(_ln(y1, ps1_ref[...], pB1_ref[...]), 0.0)
    x_ref[...] = jnp.concatenate([y0, y1], axis=1)


def _projector(h, pw0, pb0, ps0, pB0, pw1, pb1, ps1, pB1):
    full64 = pl.BlockSpec((64, 64), lambda i: (0, 0))
    vec64 = pl.BlockSpec((1, 64), lambda i: (0, 0))
    return pl.pallas_call(
        _proj_body,
        grid=(N_NODES // RB,),
        in_specs=[pl.BlockSpec((RB, D), lambda i: (i, 0)),
                  full64, vec64, vec64, vec64,
                  full64, vec64, vec64, vec64],
        out_specs=pl.BlockSpec((RB, D), lambda i: (i, 0)),
        out_shape=jax.ShapeDtypeStruct((N_NODES, D), jnp.float32),
    )(h, pw0,
      pb0.reshape(1, 64), ps0.reshape(1, 64), pB0.reshape(1, 64),
      pw1,
      pb1.reshape(1, 64), ps1.reshape(1, 64), pB1.reshape(1, 64))


def _msgpass(x, src3, dst3, zrows, zdeg, with_deg):
    mesh = plsc.VectorSubcoreMesh(core_axis_name="c", subcore_axis_name="s")

    ah_t = jax.ShapeDtypeStruct((NPAD, D), jnp.float32)
    dg_t = jax.ShapeDtypeStruct((NPAD,), jnp.float32)
    out_type = (ah_t, ah_t, dg_t, dg_t) if with_deg else (ah_t, ah_t)

    @functools.partial(
        pl.kernel,
        mesh=mesh,
        out_type=out_type,
        scratch_types=[
            pltpu.VMEM((IB, CH), jnp.int32),
            pltpu.VMEM((IB, CH), jnp.int32),
            pltpu.VMEM((CH, D), jnp.float32),
            pltpu.VMEM((CH, D), jnp.float32),
            pltpu.VMEM((CH, D), jnp.float32),
            pltpu.VMEM((CH,), jnp.float32),
            pltpu.VMEM_SHARED((NPAD, D), jnp.float32),
            pltpu.VMEM_SHARED((NPAD,), jnp.float32),
            pltpu.SemaphoreType.DMA,
            pltpu.SemaphoreType.DMA,
            pltpu.SemaphoreType.DMA,
            pltpu.SemaphoreType.DMA,
            pltpu.SemaphoreType.DMA,
            pltpu.SemaphoreType.DMA,
            pltpu.SemaphoreType.DMA,
        ],
    )
    def k(x_hbm, src_hbm, dst_hbm, zr_hbm, zd_hbm, *rest):
        if with_deg:
            (ah0_out, ah1_out, dg0_out, dg1_out,
             srcb, dstb, rows0, rows1, rows2, ones, acc, dacc,
             gs0, gs1, gs2, ss0, ss1, ss2, dsem) = rest
        else:
            (ah0_out, ah1_out,
             srcb, dstb, rows0, rows1, rows2, ones, acc, dacc,
             gs0, gs1, gs2, ss0, ss1, ss2, dsem) = rest
        c = lax.axis_index("c")
        s = lax.axis_index("s")
        wid = s * 2 + c
        r0 = pl.multiple_of(s * RPT, RPT)
        pltpu.sync_copy(zr_hbm.at[pl.ds(r0, RPT)], acc.at[pl.ds(r0, RPT)])
        if with_deg:
            pltpu.sync_copy(zd_hbm.at[pl.ds(r0, RPT)], dacc.at[pl.ds(r0, RPT)])
        if with_deg:
            for j in range(CH // 16):
                ones[pl.ds(j * 16, 16)] = jnp.full((16,), 1.0, jnp.float32)
        plsc.subcore_barrier()

        def start_g(j, rows, sem):
            pltpu.async_copy(x_hbm.at[srcb.at[j]], rows, sem)

        def wait_g(rows, sem):
            pltpu.make_async_copy(x_hbm.at[srcb.at[0]], rows, sem).wait()

        def start_s(j, rows, sem):
            pltpu.async_copy(rows, acc.at[dstb.at[j]], sem, add=True)
            if with_deg:
                pltpu.async_copy(ones, dacc.at[dstb.at[j]], dsem, add=True)

        def wait_s(rows, sem):
            pltpu.make_async_copy(rows, acc.at[dstb.at[0]], sem).wait()

        def wait_d(j, carry):
            pltpu.make_async_copy(ones, dacc.at[dstb.at[0]], dsem).wait()
            return carry

        def group(g, carry):
            pltpu.sync_copy(src_hbm.at[wid, g], srcb)
            pltpu.sync_copy(dst_hbm.at[wid, g], dstb)
            start_g(0, rows0, gs0)
            start_g(1, rows1, gs1)
            start_g(2, rows2, gs2)
            wait_g(rows0, gs0)
            start_s(0, rows0, ss0)
            wait_g(rows1, gs1)
            start_s(1, rows1, ss1)

            def body(i, carry2):
                j = i * 3 + 2
                wait_g(rows2, gs2)
                start_s(j, rows2, ss2)
                wait_s(rows0, ss0)
                start_g(j + 1, rows0, gs0)
                wait_g(rows0, gs0)
                start_s(j + 1, rows0, ss0)
                wait_s(rows1, ss1)
                start_g(j + 2, rows1, gs1)
                wait_g(rows1, gs1)
                start_s(j + 2, rows1, ss1)
                wait_s(rows2, ss2)
                start_g(j + 3, rows2, gs2)
                return carry2

            lax.fori_loop(0, (IB - 4) // 3, body, 0)
            wait_g(rows2, gs2)
            start_s(IB - 2, rows2, ss2)
            wait_s(rows0, ss0)
            start_g(IB - 1, rows0, gs0)
            wait_g(rows0, gs0)
            start_s(IB - 1, rows0, ss0)
            wait_s(rows1, ss1)
            wait_s(rows2, ss2)
            wait_s(rows0, ss0)
            if with_deg:
                lax.fori_loop(0, IB, wait_d, 0)
            return carry

        lax.fori_loop(0, NG, group, 0)
        plsc.subcore_barrier()

        @pl.when(c == 0)
        def _():
            pltpu.sync_copy(acc.at[pl.ds(r0, RPT)], ah0_out.at[pl.ds(r0, RPT)])
            if with_deg:
                pltpu.sync_copy(dacc.at[pl.ds(r0, RPT)], dg0_out.at[pl.ds(r0, RPT)])

        @pl.when(c == 1)
        def _():
            pltpu.sync_copy(acc.at[pl.ds(r0, RPT)], ah1_out.at[pl.ds(r0, RPT)])
            if with_deg:
                pltpu.sync_copy(dacc.at[pl.ds(r0, RPT)], dg1_out.at[pl.ds(r0, RPT)])

    return k(x, src3, dst3, zrows, zdeg)


def _sage_body(use_ln, use_act,
               x_ref, ah0_ref, ah1_ref, dg0_ref, dg1_ref,
               w_ref, b_ref, s_ref, bb_ref, out_ref):
    xb = x_ref[...]
    ah = ah0_ref[...] + ah1_ref[...]
    d = dg0_ref[...][:, 0] + dg1_ref[...][:, 0]
    norm = jnp.where(d > 0, 1.0 / d, 0.0)[:, None]
    m = ah * norm
    out = (jnp.dot(xb, w_ref[0], preferred_element_type=jnp.float32)
           + jnp.dot(m, w_ref[1], preferred_element_type=jnp.float32)
           + b_ref[...])
    if use_ln:
        out = _ln(out, s_ref[...], bb_ref[...])
    if use_act:
        out = jnp.maximum(out, 0.0)
    out_ref[...] = out


def _sage_dense(x, ah0, ah1, dg0, dg1, W, b, s, bb, n_out, use_ln, use_act):
    vec = pl.BlockSpec((1, n_out), lambda i: (0, 0))
    pad_rows = pl.BlockSpec((RB, D), lambda i: (i, 0))
    pad_deg = pl.BlockSpec((RB, 1), lambda i: (i, 0))
    return pl.pallas_call(
        functools.partial(_sage_body, use_ln, use_act),
        grid=(N_NODES // RB,),
        in_specs=[pl.BlockSpec((RB, D), lambda i: (i, 0)),
                  pad_rows, pad_rows, pad_deg, pad_deg,
                  pl.BlockSpec((2, D, n_out), lambda i: (0, 0, 0)),
                  vec, vec, vec],
        out_specs=pl.BlockSpec((RB, n_out), lambda i: (i, 0)),
        out_shape=jax.ShapeDtypeStruct((N_NODES, n_out), jnp.float32),
    )(x, ah0, ah1, dg0.reshape(NPAD, 1), dg1.reshape(NPAD, 1),
      W.reshape(2, D, n_out),
      b.reshape(1, n_out), s.reshape(1, n_out), bb.reshape(1, n_out))


def kernel(h, edge_index, pW0, pb0, ps0, pB0, pW1, pb1, ps1, pB1,
           W0, b0, s0, B0, W1, b1):
    ei = edge_index.astype(jnp.int32)
    src3 = ei[0].reshape(NW, NG, IB, CH)
    dst3 = ei[1].reshape(NW, NG, IB, CH)
    zrows = jnp.zeros((NPAD, D), jnp.float32)
    zdeg = jnp.zeros((NPAD,), jnp.float32)

    x = _projector(h, pW0, pb0, ps0, pB0, pW1, pb1, ps1, pB1)
    ah0, ah1, dg0, dg1 = _msgpass(x, src3, dst3, zrows, zdeg, True)
    x1 = _sage_dense(x, ah0, ah1, dg0, dg1, W0, b0, s0, B0, D, True, True)
    ah0b, ah1b, _, _ = _msgpass(x1, src3, dst3, zrows, zdeg, True)
    out = _sage_dense(x1, ah0b, ah1b, dg0, dg1, W1, b1, b1, b1,
                      NCLS, False, False)
    return out


# speedup vs baseline: 1.1920x; 1.1920x over previous
import functools

import jax
import jax.numpy as jnp
from jax import lax
from jax.experimental import pallas as pl
from jax.experimental.pallas import tpu as pltpu
from jax.experimental.pallas import tpu_sc as plsc

N_NODES = 10000
N_EDGES = 320000
D = 128
NCLS = 16

NW = 32
EPW = N_EDGES // NW
CH = 80
NCH = EPW // CH
NG = 5
IB = NCH // NG
NPAD = 10240
RPT = NPAD // 16

RB = 1000
_EPS = 1e-5


def _ln(y, s, b):
    mu = jnp.mean(y, axis=-1, keepdims=True)
    yc = y - mu
    var = jnp.mean(yc * yc, axis=-1, keepdims=True)
    return yc / jnp.sqrt(var + _EPS) * s + b


def _proj_body(h_ref, pw0_ref, pb0_ref, ps0_ref, pB0_ref,
               pw1_ref, pb1_ref, ps1_ref, pB1_ref, x_ref):
    hb = h_ref[...]
    y0 = jnp.dot(hb[:, :64], pw0_ref[...], preferred_element_type=jnp.float32) + pb0_ref[...]
    y1 = jnp.dot(hb[:, 64:], pw1_ref[...], preferred_element_type=jnp.float32) + pb1_ref[...]
    y0 = jnp.maximum(_ln(y0, ps0_ref[...], pB0_ref[...]), 0.0)
    y1 = jnp.maximum(_ln(y1, ps1_ref[...], pB1_ref[...]), 0.0)
    x_ref[...] = jnp.concatenate([y0, y1], axis=1)


def _projector(h, pw0, pb0, ps0, pB0, pw1, pb1, ps1, pB1):
    full64 = pl.BlockSpec((64, 64), lambda i: (0, 0))
    vec64 = pl.BlockSpec((1, 64), lambda i: (0, 0))
    return pl.pallas_call(
        _proj_body,
        grid=(N_NODES // RB,),
        in_specs=[pl.BlockSpec((RB, D), lambda i: (i, 0)),
                  full64, vec64, vec64, vec64,
                  full64, vec64, vec64, vec64],
        out_specs=pl.BlockSpec((RB, D), lambda i: (i, 0)),
        out_shape=jax.ShapeDtypeStruct((N_NODES, D), jnp.float32),
    )(h, pw0,
      pb0.reshape(1, 64), ps0.reshape(1, 64), pB0.reshape(1, 64),
      pw1,
      pb1.reshape(1, 64), ps1.reshape(1, 64), pB1.reshape(1, 64))


def _msgpass(x, src3, dst3, zrows, zdeg, with_deg):
    mesh = plsc.VectorSubcoreMesh(core_axis_name="c", subcore_axis_name="s")

    ah_t = jax.ShapeDtypeStruct((NPAD, D), jnp.float32)
    dg_t = jax.ShapeDtypeStruct((NPAD,), jnp.float32)
    out_type = (ah_t, ah_t, dg_t, dg_t) if with_deg else (ah_t, ah_t)

    @functools.partial(
        pl.kernel,
        mesh=mesh,
        out_type=out_type,
        scratch_types=[
            pltpu.VMEM((IB, CH), jnp.int32),
            pltpu.VMEM((IB, CH), jnp.int32),
            pltpu.VMEM((CH, D), jnp.float32),
            pltpu.VMEM((CH, D), jnp.float32),
            pltpu.VMEM((CH,), jnp.float32),
            pltpu.VMEM_SHARED((NPAD, D), jnp.float32),
            pltpu.VMEM_SHARED((NPAD,), jnp.float32),
            pltpu.SemaphoreType.DMA,
            pltpu.SemaphoreType.DMA,
            pltpu.SemaphoreType.DMA,
        ],
    )
    def k(x_hbm, src_hbm, dst_hbm, zr_hbm, zd_hbm, *rest):
        if with_deg:
            (ah0_out, ah1_out, dg0_out, dg1_out,
             srcb, dstb, rows0, rows1, ones, acc, dacc,
             gs0, gs1, dsem) = rest
        else:
            (ah0_out, ah1_out,
             srcb, dstb, rows0, rows1, ones, acc, dacc,
             gs0, gs1, dsem) = rest
        c = lax.axis_index("c")
        s = lax.axis_index("s")
        wid = s * 2 + c
        r0 = pl.multiple_of(s * RPT, RPT)
        pltpu.sync_copy(zr_hbm.at[pl.ds(r0, RPT)], acc.at[pl.ds(r0, RPT)])
        if with_deg:
            pltpu.sync_copy(zd_hbm.at[pl.ds(r0, RPT)], dacc.at[pl.ds(r0, RPT)])
        if with_deg:
            for j in range(CH // 16):
                ones[pl.ds(j * 16, 16)] = jnp.full((16,), 1.0, jnp.float32)
        plsc.subcore_barrier()

        def start_g(j, rows, sem):
            pltpu.async_copy(x_hbm.at[srcb.at[j]], rows, sem)

        def wait_g(rows, sem):
            pltpu.make_async_copy(x_hbm.at[srcb.at[0]], rows, sem).wait()

        def wait_d(j, carry):
            pltpu.make_async_copy(ones, dacc.at[dstb.at[0]], dsem).wait()
            return carry

        def scat(j, rows):
            pltpu.sync_copy(rows, acc.at[dstb.at[j]], add=True)
            if with_deg:
                pltpu.async_copy(ones, dacc.at[dstb.at[j]], dsem, add=True)

        def group(g, carry):
            pltpu.sync_copy(src_hbm.at[wid, g], srcb)
            pltpu.sync_copy(dst_hbm.at[wid, g], dstb)
            start_g(0, rows0, gs0)

            def body(i, carry2):
                j0 = i * 2
                start_g(j0 + 1, rows1, gs1)
                wait_g(rows0, gs0)
                scat(j0, rows0)
                start_g(j0 + 2, rows0, gs0)
                wait_g(rows1, gs1)
                scat(j0 + 1, rows1)
                return carry2

            lax.fori_loop(0, (IB - 1) // 2, body, 0)
            wait_g(rows0, gs0)
            scat(IB - 1, rows0)
            if with_deg:
                lax.fori_loop(0, IB, wait_d, 0)
            return carry

        lax.fori_loop(0, NG, group, 0)
        plsc.subcore_barrier()

        @pl.when(c == 0)
        def _():
            pltpu.sync_copy(acc.at[pl.ds(r0, RPT)], ah0_out.at[pl.ds(r0, RPT)])
            if with_deg:
                pltpu.sync_copy(dacc.at[pl.ds(r0, RPT)], dg0_out.at[pl.ds(r0, RPT)])

        @pl.when(c == 1)
        def _():
            pltpu.sync_copy(acc.at[pl.ds(r0, RPT)], ah1_out.at[pl.ds(r0, RPT)])
            if with_deg:
                pltpu.sync_copy(dacc.at[pl.ds(r0, RPT)], dg1_out.at[pl.ds(r0, RPT)])

    return k(x, src3, dst3, zrows, zdeg)


def _sage_body(use_ln, use_act,
               x_ref, ah0_ref, ah1_ref, dg0_ref, dg1_ref,
               w_ref, b_ref, s_ref, bb_ref, out_ref):
    xb = x_ref[...]
    ah = ah0_ref[...] + ah1_ref[...]
    d = dg0_ref[...][:, 0] + dg1_ref[...][:, 0]
    norm = jnp.where(d > 0, 1.0 / d, 0.0)[:, None]
    m = ah * norm
    out = (jnp.dot(xb, w_ref[0], preferred_element_type=jnp.float32)
           + jnp.dot(m, w_ref[1], preferred_element_type=jnp.float32)
           + b_ref[...])
    if use_ln:
        out = _ln(out, s_ref[...], bb_ref[...])
    if use_act:
        out = jnp.maximum(out, 0.0)
    out_ref[...] = out


def _sage_dense(x, ah0, ah1, dg0, dg1, W, b, s, bb, n_out, use_ln, use_act):
    vec = pl.BlockSpec((1, n_out), lambda i: (0, 0))
    pad_rows = pl.BlockSpec((RB, D), lambda i: (i, 0))
    pad_deg = pl.BlockSpec((RB, 1), lambda i: (i, 0))
    return pl.pallas_call(
        functools.partial(_sage_body, use_ln, use_act),
        grid=(N_NODES // RB,),
        in_specs=[pl.BlockSpec((RB, D), lambda i: (i, 0)),
                  pad_rows, pad_rows, pad_deg, pad_deg,
                  pl.BlockSpec((2, D, n_out), lambda i: (0, 0, 0)),
                  vec, vec, vec],
        out_specs=pl.BlockSpec((RB, n_out), lambda i: (i, 0)),
        out_shape=jax.ShapeDtypeStruct((N_NODES, n_out), jnp.float32),
    )(x, ah0, ah1, dg0.reshape(NPAD, 1), dg1.reshape(NPAD, 1),
      W.reshape(2, D, n_out),
      b.reshape(1, n_out), s.reshape(1, n_out), bb.reshape(1, n_out))


def kernel(h, edge_index, pW0, pb0, ps0, pB0, pW1, pb1, ps1, pB1,
           W0, b0, s0, B0, W1, b1):
    ei = edge_index.astype(jnp.int32)
    src3 = ei[0].reshape(NW, NG, IB, CH)
    dst3 = ei[1].reshape(NW, NG, IB, CH)
    zrows = jnp.zeros((NPAD, D), jnp.float32)
    zdeg = jnp.zeros((NPAD,), jnp.float32)

    x = _projector(h, pW0, pb0, ps0, pB0, pW1, pb1, ps1, pB1)
    ah0, ah1, dg0, dg1 = _msgpass(x, src3, dst3, zrows, zdeg, True)
    x1 = _sage_dense(x, ah0, ah1, dg0, dg1, W0, b0, s0, B0, D, True, True)
    ah0b, ah1b, _, _ = _msgpass(x1, src3, dst3, zrows, zdeg, True)
    out = _sage_dense(x1, ah0b, ah1b, dg0, dg1, W1, b1, b1, b1,
                      NCLS, False, False)
    return out


# speedup vs baseline: 1.3341x; 1.1192x over previous
import functools

import jax
import jax.numpy as jnp
from jax import lax
from jax.experimental import pallas as pl
from jax.experimental.pallas import tpu as pltpu
from jax.experimental.pallas import tpu_sc as plsc

N_NODES = 10000
N_EDGES = 320000
D = 128
NCLS = 16

NW = 32
EPW = N_EDGES // NW
CH = 80
NCH = EPW // CH
NG = 5
IB = NCH // NG
NPAD = 10240
RPT = NPAD // 16

RB = 1000
_EPS = 1e-5


def _ln(y, s, b):
    mu = jnp.mean(y, axis=-1, keepdims=True)
    yc = y - mu
    var = jnp.mean(yc * yc, axis=-1, keepdims=True)
    return yc / jnp.sqrt(var + _EPS) * s + b


def _proj_body(h_ref, pw0_ref, pb0_ref, ps0_ref, pB0_ref,
               pw1_ref, pb1_ref, ps1_ref, pB1_ref, x_ref):
    hb = h_ref[...]
    y0 = jnp.dot(hb[:, :64], pw0_ref[...], preferred_element_type=jnp.float32) + pb0_ref[...]
    y1 = jnp.dot(hb[:, 64:], pw1_ref[...], preferred_element_type=jnp.float32) + pb1_ref[...]
    y0 = jnp.maximum(_ln(y0, ps0_ref[...], pB0_ref[...]), 0.0)
    y1 = jnp.maximum(_ln(y1, ps1_ref[...], pB1_ref[...]), 0.0)
    x_ref[...] = jnp.concatenate([y0, y1], axis=1)


def _projector(h, pw0, pb0, ps0, pB0, pw1, pb1, ps1, pB1):
    full64 = pl.BlockSpec((64, 64), lambda i: (0, 0))
    vec64 = pl.BlockSpec((1, 64), lambda i: (0, 0))
    return pl.pallas_call(
        _proj_body,
        grid=(N_NODES // RB,),
        in_specs=[pl.BlockSpec((RB, D), lambda i: (i, 0)),
                  full64, vec64, vec64, vec64,
                  full64, vec64, vec64, vec64],
        out_specs=pl.BlockSpec((RB, D), lambda i: (i, 0)),
        out_shape=jax.ShapeDtypeStruct((N_NODES, D), jnp.float32),
    )(h, pw0,
      pb0.reshape(1, 64), ps0.reshape(1, 64), pB0.reshape(1, 64),
      pw1,
      pb1.reshape(1, 64), ps1.reshape(1, 64), pB1.reshape(1, 64))


def _msgpass(x, src3, dst3, zrows, zdeg, with_deg):
    mesh = plsc.VectorSubcoreMesh(core_axis_name="c", subcore_axis_name="s")

    ah_t = jax.ShapeDtypeStruct((NPAD, D), jnp.float32)
    dg_t = jax.ShapeDtypeStruct((NPAD,), jnp.float32)
    out_type = (ah_t, ah_t, dg_t, dg_t) if with_deg else (ah_t, ah_t)

    @functools.partial(
        pl.kernel,
        mesh=mesh,
        out_type=out_type,
        scratch_types=[
            pltpu.VMEM((IB, CH), jnp.int32),
            pltpu.VMEM((IB, CH), jnp.int32),
            pltpu.VMEM((CH, D), jnp.float32),
            pltpu.VMEM((CH, D), jnp.float32),
            pltpu.VMEM((CH, D), jnp.float32),
            pltpu.VMEM((CH,), jnp.float32),
            pltpu.VMEM_SHARED((NPAD, D), jnp.float32),
            pltpu.VMEM_SHARED((NPAD,), jnp.float32),
            pltpu.SemaphoreType.DMA,
            pltpu.SemaphoreType.DMA,
            pltpu.SemaphoreType.DMA,
            pltpu.SemaphoreType.DMA,
        ],
    )
    def k(x_hbm, src_hbm, dst_hbm, zr_hbm, zd_hbm, *rest):
        if with_deg:
            (ah0_out, ah1_out, dg0_out, dg1_out,
             srcb, dstb, rows0, rows1, rows2, ones, acc, dacc,
             gs0, gs1, gs2, dsem) = rest
        else:
            (ah0_out, ah1_out,
             srcb, dstb, rows0, rows1, rows2, ones, acc, dacc,
             gs0, gs1, gs2, dsem) = rest
        c = lax.axis_index("c")
        s = lax.axis_index("s")
        wid = s * 2 + c
        r0 = pl.multiple_of(s * RPT, RPT)
        pltpu.sync_copy(zr_hbm.at[pl.ds(r0, RPT)], acc.at[pl.ds(r0, RPT)])
        if with_deg:
            pltpu.sync_copy(zd_hbm.at[pl.ds(r0, RPT)], dacc.at[pl.ds(r0, RPT)])
        if with_deg:
            for j in range(CH // 16):
                ones[pl.ds(j * 16, 16)] = jnp.full((16,), 1.0, jnp.float32)
        plsc.subcore_barrier()

        def start_g(j, rows, sem):
            pltpu.async_copy(x_hbm.at[srcb.at[j]], rows, sem)

        def wait_g(rows, sem):
            pltpu.make_async_copy(x_hbm.at[srcb.at[0]], rows, sem).wait()

        def wait_d(j, carry):
            pltpu.make_async_copy(ones, dacc.at[dstb.at[0]], dsem).wait()
            return carry

        def scat(j, rows):
            pltpu.sync_copy(rows, acc.at[dstb.at[j]], add=True)
            if with_deg:
                pltpu.async_copy(ones, dacc.at[dstb.at[j]], dsem, add=True)

        def group(g, carry):
            pltpu.sync_copy(src_hbm.at[wid, g], srcb)
            pltpu.sync_copy(dst_hbm.at[wid, g], dstb)
            start_g(0, rows0, gs0)
            start_g(1, rows1, gs1)

            def body(i, carry2):
                j = i * 3
                start_g(j + 2, rows2, gs2)
                wait_g(rows0, gs0)
                scat(j, rows0)
                start_g(j + 3, rows0, gs0)
                wait_g(rows1, gs1)
                scat(j + 1, rows1)
                start_g(j + 4, rows1, gs1)
                wait_g(rows2, gs2)
                scat(j + 2, rows2)
                return carry2

            lax.fori_loop(0, (IB - 4) // 3, body, 0)
            j = IB - 4
            start_g(j + 2, rows2, gs2)
            wait_g(rows0, gs0)
            scat(j, rows0)
            start_g(j + 3, rows0, gs0)
            wait_g(rows1, gs1)
            scat(j + 1, rows1)
            wait_g(rows2, gs2)
            scat(j + 2, rows2)
            wait_g(rows0, gs0)
            scat(j + 3, rows0)
            if with_deg:
                lax.fori_loop(0, IB, wait_d, 0)
            return carry

        lax.fori_loop(0, NG, group, 0)
        plsc.subcore_barrier()

        @pl.when(c == 0)
        def _():
            pltpu.sync_copy(acc.at[pl.ds(r0, RPT)], ah0_out.at[pl.ds(r0, RPT)])
            if with_deg:
                pltpu.sync_copy(dacc.at[pl.ds(r0, RPT)], dg0_out.at[pl.ds(r0, RPT)])

        @pl.when(c == 1)
        def _():
            pltpu.sync_copy(acc.at[pl.ds(r0, RPT)], ah1_out.at[pl.ds(r0, RPT)])
            if with_deg:
                pltpu.sync_copy(dacc.at[pl.ds(r0, RPT)], dg1_out.at[pl.ds(r0, RPT)])

    return k(x, src3, dst3, zrows, zdeg)


def _sage_body(use_ln, use_act,
               x_ref, ah0_ref, ah1_ref, dg0_ref, dg1_ref,
               w_ref, b_ref, s_ref, bb_ref, out_ref):
    xb = x_ref[...]
    ah = ah0_ref[...] + ah1_ref[...]
    d = dg0_ref[...][:, 0] + dg1_ref[...][:, 0]
    norm = jnp.where(d > 0, 1.0 / d, 0.0)[:, None]
    m = ah * norm
    out = (jnp.dot(xb, w_ref[0], preferred_element_type=jnp.float32)
           + jnp.dot(m, w_ref[1], preferred_element_type=jnp.float32)
           + b_ref[...])
    if use_ln:
        out = _ln(out, s_ref[...], bb_ref[...])
    if use_act:
        out = jnp.maximum(out, 0.0)
    out_ref[...] = out


def _sage_dense(x, ah0, ah1, dg0, dg1, W, b, s, bb, n_out, use_ln, use_act):
    vec = pl.BlockSpec((1, n_out), lambda i: (0, 0))
    pad_rows = pl.BlockSpec((RB, D), lambda i: (i, 0))
    pad_deg = pl.BlockSpec((RB, 1), lambda i: (i, 0))
    return pl.pallas_call(
        functools.partial(_sage_body, use_ln, use_act),
        grid=(N_NODES // RB,),
        in_specs=[pl.BlockSpec((RB, D), lambda i: (i, 0)),
                  pad_rows, pad_rows, pad_deg, pad_deg,
                  pl.BlockSpec((2, D, n_out), lambda i: (0, 0, 0)),
                  vec, vec, vec],
        out_specs=pl.BlockSpec((RB, n_out), lambda i: (i, 0)),
        out_shape=jax.ShapeDtypeStruct((N_NODES, n_out), jnp.float32),
    )(x, ah0, ah1, dg0.reshape(NPAD, 1), dg1.reshape(NPAD, 1),
      W.reshape(2, D, n_out),
      b.reshape(1, n_out), s.reshape(1, n_out), bb.reshape(1, n_out))


def kernel(h, edge_index, pW0, pb0, ps0, pB0, pW1, pb1, ps1, pB1,
           W0, b0, s0, B0, W1, b1):
    ei = edge_index.astype(jnp.int32)
    src3 = ei[0].reshape(NW, NG, IB, CH)
    dst3 = ei[1].reshape(NW, NG, IB, CH)
    zrows = jnp.zeros((NPAD, D), jnp.float32)
    zdeg = jnp.zeros((NPAD,), jnp.float32)

    x = _projector(h, pW0, pb0, ps0, pB0, pW1, pb1, ps1, pB1)
    ah0, ah1, dg0, dg1 = _msgpass(x, src3, dst3, zrows, zdeg, True)
    x1 = _sage_dense(x, ah0, ah1, dg0, dg1, W0, b0, s0, B0, D, True, True)
    ah0b, ah1b, _, _ = _msgpass(x1, src3, dst3, zrows, zdeg, True)
    out = _sage_dense(x1, ah0b, ah1b, dg0, dg1, W1, b1, b1, b1,
                      NCLS, False, False)
    return out
